# baseline (speedup 1.0000x reference)
import jax
import jax.numpy as jnp
from jax import lax
from jax.experimental import pallas as pl
from jax.experimental.pallas import tpu as pltpu
from jax.experimental.pallas import tpu_sc as plsc

_VOCAB = 1000
_EMBED = 64
_BATCH = 16384

_NC = 2
_NS = 16
_NW = _NC * _NS
_BPW = _BATCH // _NW
_RING = 10
_L = 16


def _gather_body(idx0_hbm, idx1_hbm, table_hbm, out_hbm,
                 i0_v, i1_v, piece_v, rows_v, sem):
    wid = lax.axis_index("s") * _NC + lax.axis_index("c")
    base = wid * _BPW
    pltpu.sync_copy(idx0_hbm.at[pl.ds(base, _BPW)], i0_v.at[pl.ds(0, _BPW)])
    pltpu.sync_copy(idx1_hbm.at[pl.ds(base, _BPW)], i1_v.at[pl.ds(0, _BPW)])

    lane = lax.iota(jnp.int32, _L)

    def fire(b):
        i0 = i0_v[pl.ds(b, _L)][0]
        i1 = i1_v[pl.ds(b, _L)][0]
        ct = pl.multiple_of(lax.shift_right_logical(i1, 7) * 128, 128)
        pltpu.make_async_copy(
            table_hbm.at[i0, :, pl.ds(ct, 128)],
            piece_v.at[lax.rem(b, _RING)], sem,
        ).start()

    def wait_slot(b):
        pltpu.make_async_copy(
            table_hbm.at[0, :, pl.ds(pl.multiple_of(b * 0, 128), 128)],
            piece_v.at[lax.rem(b, _RING)], sem,
        ).wait()

    def extract(b):
        slot = lax.rem(b, _RING) + lane * 0
        cl = (i1_v[pl.ds(b, _L)] & 127)[0] + lane * 0
        dst = b * _EMBED
        for k in range(_EMBED // _L):
            val = plsc.load_gather(piece_v, [slot, lane + k * _L, cl])
            rows_v[pl.ds(dst + k * _L, _L)] = val

    def step(b, carry):
        @pl.when(b >= _RING)
        def _drain():
            wait_slot(b - _RING)
            extract(b - _RING)

        @pl.when(b < _BPW)
        def _fire():
            fire(b)

        return carry

    lax.fori_loop(0, _BPW + _RING, step, 0)

    pltpu.sync_copy(rows_v, out_hbm.at[pl.ds(base * _EMBED, _BPW * _EMBED)])


@jax.jit
def kernel(idx0, idx1, lookup):
    table_t = jnp.transpose(lookup, (0, 2, 1))
    run = pl.kernel(
        _gather_body,
        out_type=jax.ShapeDtypeStruct((_BATCH * _EMBED,), jnp.float32),
        mesh=plsc.VectorSubcoreMesh(core_axis_name="c", subcore_axis_name="s"),
        compiler_params=pltpu.CompilerParams(
            use_tc_tiling_on_sc=True, needs_layout_passes=False),
        scratch_types=[
            pltpu.VMEM((_BPW + _L,), jnp.int32),
            pltpu.VMEM((_BPW + _L,), jnp.int32),
            pltpu.VMEM((_RING, _EMBED, 128), jnp.float32),
            pltpu.VMEM((_BPW * _EMBED,), jnp.float32),
            pltpu.SemaphoreType.DMA,
        ],
    )
    out_flat = run(idx0, idx1, table_t)
    return out_flat.reshape(_BATCH, _EMBED)


# speedup vs baseline: 1.0181x; 1.0181x over previous
import jax
import jax.numpy as jnp
from jax import lax
from jax.experimental import pallas as pl
from jax.experimental.pallas import tpu as pltpu
from jax.experimental.pallas import tpu_sc as plsc

_VOCAB = 1000
_EMBED = 64
_BATCH = 16384

_NC = 2
_NS = 16
_NW = _NC * _NS
_BPW = _BATCH // _NW
_RING = 8
_L = 16


def _gather_body(idx0_hbm, idx1_hbm, table_hbm, out_hbm,
                 i0_v, i1_v, piece_v, rows_v, sem):
    wid = lax.axis_index("s") * _NC + lax.axis_index("c")
    base = wid * _BPW
    pltpu.sync_copy(idx0_hbm.at[pl.ds(base, _BPW)], i0_v.at[pl.ds(0, _BPW)])
    pltpu.sync_copy(idx1_hbm.at[pl.ds(base, _BPW)], i1_v.at[pl.ds(0, _BPW)])

    lane = lax.iota(jnp.int32, _L)

    def fire(b):
        i0 = i0_v[pl.ds(b, _L)][0]
        i1 = i1_v[pl.ds(b, _L)][0]
        ct = pl.multiple_of(lax.shift_right_logical(i1, 7) * 128, 128)
        pltpu.make_async_copy(
            table_hbm.at[i0, :, pl.ds(ct, 128)],
            piece_v.at[b & (_RING - 1)], sem,
        ).start()

    def wait_slot(b):
        pltpu.make_async_copy(
            table_hbm.at[0, :, pl.ds(pl.multiple_of(b * 0, 128), 128)],
            piece_v.at[b & (_RING - 1)], sem,
        ).wait()

    def extract(b):
        slot = (b & (_RING - 1)) + lane * 0
        cl = (i1_v[pl.ds(b, _L)] & 127)[0] + lane * 0
        dst = b * _EMBED
        for k in range(_EMBED // _L):
            val = plsc.load_gather(piece_v, [slot, lane + k * _L, cl])
            rows_v[pl.ds(dst + k * _L, _L)] = val

    def step(b, carry):
        @pl.when(b >= _RING)
        def _drain():
            wait_slot(b - _RING)
            extract(b - _RING)

        @pl.when(b < _BPW)
        def _fire():
            fire(b)

        return carry

    lax.fori_loop(0, _BPW + _RING, step, 0)

    pltpu.sync_copy(rows_v, out_hbm.at[pl.ds(base * _EMBED, _BPW * _EMBED)])


@jax.jit
def kernel(idx0, idx1, lookup):
    table_t = jnp.transpose(lookup, (0, 2, 1))
    run = pl.kernel(
        _gather_body,
        out_type=jax.ShapeDtypeStruct((_BATCH * _EMBED,), jnp.float32),
        mesh=plsc.VectorSubcoreMesh(core_axis_name="c", subcore_axis_name="s"),
        compiler_params=pltpu.CompilerParams(
            use_tc_tiling_on_sc=True, needs_layout_passes=False),
        scratch_types=[
            pltpu.VMEM((_BPW + _L,), jnp.int32),
            pltpu.VMEM((_BPW + _L,), jnp.int32),
            pltpu.VMEM((_RING, _EMBED, 128), jnp.float32),
            pltpu.VMEM((_BPW * _EMBED,), jnp.float32),
            pltpu.SemaphoreType.DMA,
        ],
    )
    out_flat = run(idx0, idx1, table_t)
    return out_flat.reshape(_BATCH, _EMBED)


# speedup vs baseline: 1.0482x; 1.0295x over previous
import jax
import jax.numpy as jnp
from jax import lax
from jax.experimental import pallas as pl
from jax.experimental.pallas import tpu as pltpu
from jax.experimental.pallas import tpu_sc as plsc

_VOCAB = 1000
_EMBED = 64
_BATCH = 16384

_NC = 2
_NS = 16
_NW = _NC * _NS
_BPW = _BATCH // _NW
_RING = 8
_L = 16


def _gather_body(idx0_hbm, idx1_hbm, table_hbm, out_hbm,
                 i0_v, i1_v, piece_v, rows2_v, sem, sem2):
    wid = lax.axis_index("s") * _NC + lax.axis_index("c")
    base = wid * _BPW
    pltpu.sync_copy(idx0_hbm.at[pl.ds(base, _BPW)], i0_v.at[pl.ds(0, _BPW)])
    pltpu.sync_copy(idx1_hbm.at[pl.ds(base, _BPW)], i1_v.at[pl.ds(0, _BPW)])

    lane = lax.iota(jnp.int32, _L)

    def fire(b):
        i0 = i0_v[pl.ds(b, _L)][0]
        i1 = i1_v[pl.ds(b, _L)][0]
        ct = pl.multiple_of(lax.shift_right_logical(i1, 7) * 128, 128)
        pltpu.make_async_copy(
            table_hbm.at[i0, :, pl.ds(ct, 128)],
            piece_v.at[b & (_RING - 1)], sem,
        ).start()

    def wait_slot(b):
        pltpu.make_async_copy(
            table_hbm.at[0, :, pl.ds(pl.multiple_of(b * 0, 128), 128)],
            piece_v.at[b & (_RING - 1)], sem,
        ).wait()

    def out_desc(grp, half):
        return pltpu.make_async_copy(
            rows2_v.at[half], out_hbm.at[pl.ds(base + grp * 64, 64)], sem2
        )

    def extract(b):
        slot = (b & (_RING - 1)) + lane * 0
        cl = (i1_v[pl.ds(b, _L)] & 127)[0] + lane * 0
        half = lax.shift_right_logical(b, 6) & 1
        row = b & 63
        for k in range(_EMBED // _L):
            val = plsc.load_gather(piece_v, [slot, lane + k * _L, cl])
            rows2_v[half, row, pl.ds(k * _L, _L)] = val

    def step(b, carry):
        @pl.when(b >= _RING)
        def _drain():
            e = b - _RING
            @pl.when((e >= 128) & ((e & 63) == 0))
            def _wait_half():
                g_old = lax.shift_right_logical(e, 6) - 2
                out_desc(g_old, lax.shift_right_logical(e, 6) & 1).wait()

            wait_slot(e)
            extract(e)

            @pl.when((e & 63) == 63)
            def _flush():
                g = lax.shift_right_logical(e, 6)
                out_desc(g, g & 1).start()

        @pl.when(b < _BPW)
        def _fire():
            fire(b)

        return carry

    lax.fori_loop(0, _BPW + _RING, step, 0)
    ngrp = _BPW // 64

    def drain_out(g, carry):
        out_desc(g, g & 1).wait()
        return carry

    lax.fori_loop(ngrp - 2, ngrp, drain_out, 0)


@jax.jit
def kernel(idx0, idx1, lookup):
    table_t = jnp.transpose(lookup, (0, 2, 1))
    run = pl.kernel(
        _gather_body,
        out_type=jax.ShapeDtypeStruct((_BATCH, _EMBED), jnp.float32),
        mesh=plsc.VectorSubcoreMesh(core_axis_name="c", subcore_axis_name="s"),
        compiler_params=pltpu.CompilerParams(
            use_tc_tiling_on_sc=True, needs_layout_passes=False),
        scratch_types=[
            pltpu.VMEM((_BPW + _L,), jnp.int32),
            pltpu.VMEM((_BPW + _L,), jnp.int32),
            pltpu.VMEM((_RING, _EMBED, 128), jnp.float32),
            pltpu.VMEM((2, 64, _EMBED), jnp.float32),
            pltpu.SemaphoreType.DMA,
            pltpu.SemaphoreType.DMA,
        ],
    )
    return run(idx0, idx1, table_t)
